# baseline (speedup 1.0000x reference)
import jax
import jax.numpy as jnp
from jax import lax
from jax.experimental import pallas as pl
from jax.experimental.pallas import tpu as pltpu
from jax.experimental.pallas import tpu_sc as plsc

_N = 10000
_E = 320000
_D = 128
_NC = 2
_NS = 16
_NW = _NC * _NS
_C = 128
_CHUNKS = 80
_EPW = _C * _CHUNKS
_EPAD = _EPW * _NW
_PAD = _EPAD - _E
_NACC = 10240
_RPT = _NACC // _NS

_mesh = plsc.VectorSubcoreMesh(
    core_axis_name="c", subcore_axis_name="s", num_cores=_NC, num_subcores=_NS
)


def _fill2d(ref, rows, cols, val):
    def body(i, c):
        for k in range(cols // 16):
            ref[i, pl.ds(16 * k, 16)] = jnp.full((16,), val, jnp.float32)
        return c
    lax.fori_loop(0, rows, body, 0)


def _fill1d(ref, n, val):
    def body(i, c):
        ref[pl.ds(i * 16, 16)] = jnp.full((16,), val, jnp.float32)
        return c
    lax.fori_loop(0, n // 16, body, 0)


def _zero_acc(acc_sh, buf, z0):
    for k in range(5):
        off = pl.multiple_of(z0 + k * 128, 8)
        pltpu.sync_copy(buf, acc_sh.at[pl.ds(off, 128)])


def _copy_out(acc_sh, buf, out, cid, z0):
    for k in range(5):
        off = pl.multiple_of(z0 + k * 128, 8)
        pltpu.sync_copy(acc_sh.at[pl.ds(off, 128)], buf)
        pltpu.sync_copy(buf, out.at[cid, pl.ds(off, 128)])


def _gather_scatter_loop(table, src3, dst3, acc_sh, wid,
                         s0, s1, dst_a, rows_a, rows_b, gA, gB, sA, sB,
                         deg=None):
    pltpu.sync_copy(dst3.at[wid], dst_a)
    if deg is not None:
        deg_sh, ones_v = deg
    pltpu.sync_copy(src3.at[wid, 0], s0)
    pltpu.async_copy(table.at[s0], rows_a, gA)
    pltpu.sync_copy(src3.at[wid, 1], s1)

    def body(jj, c):
        j0 = 2 * jj

        @pl.when(jj > 0)
        def _():
            pltpu.make_async_copy(rows_b, acc_sh.at[dst_a.at[0]], sB).wait()

        pltpu.async_copy(table.at[s1], rows_b, gB)
        pltpu.make_async_copy(table.at[pl.ds(0, _C)], rows_a, gA).wait()
        j2 = jnp.where(j0 + 2 < _CHUNKS, j0 + 2, 0)
        pltpu.sync_copy(src3.at[wid, j2], s0)
        pltpu.async_copy(rows_a, acc_sh.at[dst_a.at[j0]], sA, add=True)
        if deg is not None:
            pltpu.sync_copy(ones_v, deg_sh.at[dst_a.at[j0]], add=True)

        pltpu.make_async_copy(table.at[pl.ds(0, _C)], rows_b, gB).wait()
        j3 = jnp.where(j0 + 3 < _CHUNKS, j0 + 3, 0)
        pltpu.sync_copy(src3.at[wid, j3], s1)
        pltpu.async_copy(rows_b, acc_sh.at[dst_a.at[j0 + 1]], sB, add=True)
        if deg is not None:
            pltpu.sync_copy(ones_v, deg_sh.at[dst_a.at[j0 + 1]], add=True)

        pltpu.make_async_copy(rows_a, acc_sh.at[dst_a.at[0]], sA).wait()

        @pl.when(jj + 1 < _CHUNKS // 2)
        def _():
            pltpu.async_copy(table.at[s0], rows_a, gA)

        return c

    lax.fori_loop(0, _CHUNKS // 2, body, 0)
    pltpu.make_async_copy(rows_b, acc_sh.at[dst_a.at[0]], sB).wait()


def _agg_deg_body(table, src3, dst3, parts, degp,
                  s0, s1, dst_a, rows_a, rows_b, ones_v, vec_v,
                  acc_sh, deg_sh, gA, gB, sA, sB):
    cid = lax.axis_index("c")
    sid = lax.axis_index("s")
    wid = cid * _NS + sid
    z0 = pl.multiple_of(sid * _RPT, 8)
    _fill2d(rows_a, 128, _D, 0.0)
    _zero_acc(acc_sh, rows_a, z0)
    _fill1d(vec_v, _RPT, 0.0)
    pltpu.sync_copy(vec_v, deg_sh.at[pl.ds(z0, _RPT)])
    _fill1d(ones_v, _C, 1.0)
    plsc.subcore_barrier()

    _gather_scatter_loop(table, src3, dst3, acc_sh, wid,
                         s0, s1, dst_a, rows_a, rows_b, gA, gB, sA, sB,
                         deg=(deg_sh, ones_v))
    plsc.subcore_barrier()

    _copy_out(acc_sh, rows_b, parts, cid, z0)
    pltpu.sync_copy(deg_sh.at[pl.ds(z0, _RPT)], vec_v)
    pltpu.sync_copy(vec_v, degp.at[cid, pl.ds(z0, _RPT)])


def _agg_body(table, src3, dst3, parts,
              s0, s1, dst_a, rows_a, rows_b, acc_sh, gA, gB, sA, sB):
    cid = lax.axis_index("c")
    sid = lax.axis_index("s")
    wid = cid * _NS + sid
    z0 = pl.multiple_of(sid * _RPT, 8)
    _fill2d(rows_a, 128, _D, 0.0)
    _zero_acc(acc_sh, rows_a, z0)
    plsc.subcore_barrier()

    _gather_scatter_loop(table, src3, dst3, acc_sh, wid,
                         s0, s1, dst_a, rows_a, rows_b, gA, gB, sA, sB)
    plsc.subcore_barrier()

    _copy_out(acc_sh, rows_b, parts, cid, z0)


_agg_deg = pl.kernel(
    _agg_deg_body,
    out_type=(
        jax.ShapeDtypeStruct((_NC, _NACC, _D), jnp.float32),
        jax.ShapeDtypeStruct((_NC, _NACC), jnp.float32),
    ),
    mesh=_mesh,
    scratch_types=[
        pltpu.VMEM((_C,), jnp.int32),
        pltpu.VMEM((_C,), jnp.int32),
        pltpu.VMEM((_CHUNKS, _C), jnp.int32),
        pltpu.VMEM((_C, _D), jnp.float32),
        pltpu.VMEM((_C, _D), jnp.float32),
        pltpu.VMEM((_C,), jnp.float32),
        pltpu.VMEM((_RPT,), jnp.float32),
        pltpu.VMEM_SHARED((_NACC, _D), jnp.float32),
        pltpu.VMEM_SHARED((_NACC,), jnp.float32),
        pltpu.SemaphoreType.DMA,
        pltpu.SemaphoreType.DMA,
        pltpu.SemaphoreType.DMA,
        pltpu.SemaphoreType.DMA,
    ],
)

_agg = pl.kernel(
    _agg_body,
    out_type=jax.ShapeDtypeStruct((_NC, _NACC, _D), jnp.float32),
    mesh=_mesh,
    scratch_types=[
        pltpu.VMEM((_C,), jnp.int32),
        pltpu.VMEM((_C,), jnp.int32),
        pltpu.VMEM((_CHUNKS, _C), jnp.int32),
        pltpu.VMEM((_C, _D), jnp.float32),
        pltpu.VMEM((_C, _D), jnp.float32),
        pltpu.VMEM_SHARED((_NACC, _D), jnp.float32),
        pltpu.SemaphoreType.DMA,
        pltpu.SemaphoreType.DMA,
        pltpu.SemaphoreType.DMA,
        pltpu.SemaphoreType.DMA,
    ],
)


_R = 2000
_G = _N // _R

_DN = (((1,), (1,)), ((), ()))


def _dense_ln_body(parts, deg, h, Wl, bl, Wr, g, b, out):
    p = parts[0] + parts[1]
    agg = p / jnp.maximum(deg[...], 1.0)
    t = (lax.dot_general(agg, Wl[...], _DN, preferred_element_type=jnp.float32)
         + bl[...]
         + lax.dot_general(h[...], Wr[...], _DN, preferred_element_type=jnp.float32))
    mu = jnp.mean(t, axis=-1, keepdims=True)
    var = jnp.mean((t - mu) ** 2, axis=-1, keepdims=True)
    t = (t - mu) / jnp.sqrt(var + 1e-5) * g[...] + b[...]
    out[...] = jnp.maximum(t, 0.0) + h[...]


def _dense_fin_body(parts, deg, h, Wl, bl, Wr, out):
    p = parts[0] + parts[1]
    agg = p / jnp.maximum(deg[...], 1.0)
    out[...] = (lax.dot_general(agg, Wl[...], _DN, preferred_element_type=jnp.float32)
                + bl[...]
                + lax.dot_general(h[...], Wr[...], _DN, preferred_element_type=jnp.float32))


_spec_parts = pl.BlockSpec((_NC, _R, _D), lambda i: (0, i, 0))
_spec_deg = pl.BlockSpec((_R, 1), lambda i: (i, 0))
_spec_rows = pl.BlockSpec((_R, _D), lambda i: (i, 0))
_spec_w = pl.BlockSpec((_D, _D), lambda i: (0, 0))
_spec_v = pl.BlockSpec((1, _D), lambda i: (0, 0))

_dense_ln = pl.pallas_call(
    _dense_ln_body,
    grid=(_G,),
    in_specs=[_spec_parts, _spec_deg, _spec_rows, _spec_w, _spec_v,
              _spec_w, _spec_v, _spec_v],
    out_specs=_spec_rows,
    out_shape=jax.ShapeDtypeStruct((_N, _D), jnp.float32),
)

_dense_fin = pl.pallas_call(
    _dense_fin_body,
    grid=(_G,),
    in_specs=[_spec_parts, _spec_deg, _spec_rows, _spec_w, _spec_v,
              _spec_w],
    out_specs=_spec_rows,
    out_shape=jax.ShapeDtypeStruct((_N, _D), jnp.float32),
)


def kernel(x, edge_index, Wl0, bl0, Wr0, Wl1, bl1, Wr1, Wl2, bl2, Wr2,
           g0, b0, g1, b1):
    src = edge_index[0]
    dst = edge_index[1]
    ar = jnp.arange(_PAD, dtype=jnp.int32)
    src3 = jnp.concatenate([src, (ar * 37) % _N]).reshape(_NW, _CHUNKS, _C)
    dst3 = jnp.concatenate([dst, _N + (ar % (_NACC - _N))]).reshape(_NW, _CHUNKS, _C)

    bl0r, bl1r, bl2r = (v.reshape(1, _D) for v in (bl0, bl1, bl2))
    g0r, b0r, g1r, b1r = (v.reshape(1, _D) for v in (g0, b0, g1, b1))

    parts0, degp = _agg_deg(x, src3, dst3)
    deg = (degp[0, :_N] + degp[1, :_N]).reshape(_N, 1)
    h1 = _dense_ln(parts0, deg, x, Wl0, bl0r, Wr0, g0r, b0r)
    parts1 = _agg(h1, src3, dst3)
    h2 = _dense_ln(parts1, deg, h1, Wl1, bl1r, Wr1, g1r, b1r)
    parts2 = _agg(h2, src3, dst3)
    return _dense_fin(parts2, deg, h2, Wl2, bl2r, Wr2)


# speedup vs baseline: 1.0901x; 1.0901x over previous
import jax
import jax.numpy as jnp
from jax import lax
from jax.experimental import pallas as pl
from jax.experimental.pallas import tpu as pltpu
from jax.experimental.pallas import tpu_sc as plsc

_N = 10000
_E = 320000
_D = 128
_NC = 2
_NS = 16
_NW = _NC * _NS
_C = 128
_CHUNKS = 80
_EPW = _C * _CHUNKS
_EPAD = _EPW * _NW
_PAD = _EPAD - _E
_NACC = 10240
_RPT = _NACC // _NS

_mesh = plsc.VectorSubcoreMesh(
    core_axis_name="c", subcore_axis_name="s", num_cores=_NC, num_subcores=_NS
)


def _fill2d(ref, rows, cols, val):
    def body(i, c):
        for k in range(cols // 16):
            ref[i, pl.ds(16 * k, 16)] = jnp.full((16,), val, jnp.float32)
        return c
    lax.fori_loop(0, rows, body, 0)


def _fill1d(ref, n, val):
    def body(i, c):
        ref[pl.ds(i * 16, 16)] = jnp.full((16,), val, jnp.float32)
        return c
    lax.fori_loop(0, n // 16, body, 0)


def _zero_acc(acc_sh, buf, z0):
    for k in range(5):
        off = pl.multiple_of(z0 + k * 128, 8)
        pltpu.sync_copy(buf, acc_sh.at[pl.ds(off, 128)])


def _copy_out(acc_sh, buf, out, cid, z0):
    for k in range(5):
        off = pl.multiple_of(z0 + k * 128, 8)
        pltpu.sync_copy(acc_sh.at[pl.ds(off, 128)], buf)
        pltpu.sync_copy(buf, out.at[cid, pl.ds(off, 128)])


def _gather_scatter_loop(table, src3, dst3, acc_sh, wid,
                         s0, s1, dst_a, rows_a, rows_b, gA, gB,
                         deg=None):
    pltpu.sync_copy(dst3.at[wid], dst_a)
    if deg is not None:
        deg_sh, ones_v = deg
    pltpu.sync_copy(src3.at[wid, 0], s0)
    pltpu.async_copy(table.at[s0], rows_a, gA)

    def pair(jj, c):
        j0 = 2 * jj
        pltpu.sync_copy(src3.at[wid, j0 + 1], s1)
        pltpu.async_copy(table.at[s1], rows_b, gB)
        pltpu.make_async_copy(table.at[pl.ds(0, _C)], rows_a, gA).wait()
        pltpu.sync_copy(rows_a, acc_sh.at[dst_a.at[j0]], add=True)
        if deg is not None:
            pltpu.sync_copy(ones_v, deg_sh.at[dst_a.at[j0]], add=True)

        @pl.when(jj + 1 < _CHUNKS // 2)
        def _():
            pltpu.sync_copy(src3.at[wid, j0 + 2], s0)
            pltpu.async_copy(table.at[s0], rows_a, gA)

        pltpu.make_async_copy(table.at[pl.ds(0, _C)], rows_b, gB).wait()
        pltpu.sync_copy(rows_b, acc_sh.at[dst_a.at[j0 + 1]], add=True)
        if deg is not None:
            pltpu.sync_copy(ones_v, deg_sh.at[dst_a.at[j0 + 1]], add=True)
        return c

    lax.fori_loop(0, _CHUNKS // 2, pair, 0)


def _agg_deg_body(table, src3, dst3, parts, degp,
                  s0, s1, dst_a, rows_a, rows_b, ones_v, vec_v,
                  acc_sh, deg_sh, gA, gB):
    cid = lax.axis_index("c")
    sid = lax.axis_index("s")
    wid = cid * _NS + sid
    z0 = pl.multiple_of(sid * _RPT, 8)
    _fill2d(rows_a, 128, _D, 0.0)
    _zero_acc(acc_sh, rows_a, z0)
    _fill1d(vec_v, _RPT, 0.0)
    pltpu.sync_copy(vec_v, deg_sh.at[pl.ds(z0, _RPT)])
    _fill1d(ones_v, _C, 1.0)
    plsc.subcore_barrier()

    _gather_scatter_loop(table, src3, dst3, acc_sh, wid,
                         s0, s1, dst_a, rows_a, rows_b, gA, gB,
                         deg=(deg_sh, ones_v))
    plsc.subcore_barrier()

    _copy_out(acc_sh, rows_b, parts, cid, z0)
    pltpu.sync_copy(deg_sh.at[pl.ds(z0, _RPT)], vec_v)
    pltpu.sync_copy(vec_v, degp.at[cid, pl.ds(z0, _RPT)])


def _agg_body(table, src3, dst3, parts,
              s0, s1, dst_a, rows_a, rows_b, acc_sh, gA, gB):
    cid = lax.axis_index("c")
    sid = lax.axis_index("s")
    wid = cid * _NS + sid
    z0 = pl.multiple_of(sid * _RPT, 8)
    _fill2d(rows_a, 128, _D, 0.0)
    _zero_acc(acc_sh, rows_a, z0)
    plsc.subcore_barrier()

    _gather_scatter_loop(table, src3, dst3, acc_sh, wid,
                         s0, s1, dst_a, rows_a, rows_b, gA, gB)
    plsc.subcore_barrier()

    _copy_out(acc_sh, rows_b, parts, cid, z0)


_agg_deg = pl.kernel(
    _agg_deg_body,
    out_type=(
        jax.ShapeDtypeStruct((_NC, _NACC, _D), jnp.float32),
        jax.ShapeDtypeStruct((_NC, _NACC), jnp.float32),
    ),
    mesh=_mesh,
    scratch_types=[
        pltpu.VMEM((_C,), jnp.int32),
        pltpu.VMEM((_C,), jnp.int32),
        pltpu.VMEM((_CHUNKS, _C), jnp.int32),
        pltpu.VMEM((_C, _D), jnp.float32),
        pltpu.VMEM((_C, _D), jnp.float32),
        pltpu.VMEM((_C,), jnp.float32),
        pltpu.VMEM((_RPT,), jnp.float32),
        pltpu.VMEM_SHARED((_NACC, _D), jnp.float32),
        pltpu.VMEM_SHARED((_NACC,), jnp.float32),
        pltpu.SemaphoreType.DMA,
        pltpu.SemaphoreType.DMA,
    ],
)

_agg = pl.kernel(
    _agg_body,
    out_type=jax.ShapeDtypeStruct((_NC, _NACC, _D), jnp.float32),
    mesh=_mesh,
    scratch_types=[
        pltpu.VMEM((_C,), jnp.int32),
        pltpu.VMEM((_C,), jnp.int32),
        pltpu.VMEM((_CHUNKS, _C), jnp.int32),
        pltpu.VMEM((_C, _D), jnp.float32),
        pltpu.VMEM((_C, _D), jnp.float32),
        pltpu.VMEM_SHARED((_NACC, _D), jnp.float32),
        pltpu.SemaphoreType.DMA,
        pltpu.SemaphoreType.DMA,
    ],
)


_R = 2000
_G = _N // _R

_DN = (((1,), (1,)), ((), ()))


def _dense_ln_body(parts, deg, h, Wl, bl, Wr, g, b, out):
    p = parts[0] + parts[1]
    agg = p / jnp.maximum(deg[...], 1.0)
    t = (lax.dot_general(agg, Wl[...], _DN, preferred_element_type=jnp.float32)
         + bl[...]
         + lax.dot_general(h[...], Wr[...], _DN, preferred_element_type=jnp.float32))
    mu = jnp.mean(t, axis=-1, keepdims=True)
    var = jnp.mean((t - mu) ** 2, axis=-1, keepdims=True)
    t = (t - mu) / jnp.sqrt(var + 1e-5) * g[...] + b[...]
    out[...] = jnp.maximum(t, 0.0) + h[...]


def _dense_fin_body(parts, deg, h, Wl, bl, Wr, out):
    p = parts[0] + parts[1]
    agg = p / jnp.maximum(deg[...], 1.0)
    out[...] = (lax.dot_general(agg, Wl[...], _DN, preferred_element_type=jnp.float32)
                + bl[...]
                + lax.dot_general(h[...], Wr[...], _DN, preferred_element_type=jnp.float32))


_spec_parts = pl.BlockSpec((_NC, _R, _D), lambda i: (0, i, 0))
_spec_deg = pl.BlockSpec((_R, 1), lambda i: (i, 0))
_spec_rows = pl.BlockSpec((_R, _D), lambda i: (i, 0))
_spec_w = pl.BlockSpec((_D, _D), lambda i: (0, 0))
_spec_v = pl.BlockSpec((1, _D), lambda i: (0, 0))

_dense_ln = pl.pallas_call(
    _dense_ln_body,
    grid=(_G,),
    in_specs=[_spec_parts, _spec_deg, _spec_rows, _spec_w, _spec_v,
              _spec_w, _spec_v, _spec_v],
    out_specs=_spec_rows,
    out_shape=jax.ShapeDtypeStruct((_N, _D), jnp.float32),
)

_dense_fin = pl.pallas_call(
    _dense_fin_body,
    grid=(_G,),
    in_specs=[_spec_parts, _spec_deg, _spec_rows, _spec_w, _spec_v,
              _spec_w],
    out_specs=_spec_rows,
    out_shape=jax.ShapeDtypeStruct((_N, _D), jnp.float32),
)


def kernel(x, edge_index, Wl0, bl0, Wr0, Wl1, bl1, Wr1, Wl2, bl2, Wr2,
           g0, b0, g1, b1):
    src = edge_index[0]
    dst = edge_index[1]
    ar = jnp.arange(_PAD, dtype=jnp.int32)
    src3 = jnp.concatenate([src, (ar * 37) % _N]).reshape(_NW, _CHUNKS, _C)
    dst3 = jnp.concatenate([dst, _N + (ar % (_NACC - _N))]).reshape(_NW, _CHUNKS, _C)

    bl0r, bl1r, bl2r = (v.reshape(1, _D) for v in (bl0, bl1, bl2))
    g0r, b0r, g1r, b1r = (v.reshape(1, _D) for v in (g0, b0, g1, b1))

    parts0, degp = _agg_deg(x, src3, dst3)
    deg = (degp[0, :_N] + degp[1, :_N]).reshape(_N, 1)
    h1 = _dense_ln(parts0, deg, x, Wl0, bl0r, Wr0, g0r, b0r)
    parts1 = _agg(h1, src3, dst3)
    h2 = _dense_ln(parts1, deg, h1, Wl1, bl1r, Wr1, g1r, b1r)
    parts2 = _agg(h2, src3, dst3)
    return _dense_fin(parts2, deg, h2, Wl2, bl2r, Wr2)


# speedup vs baseline: 1.1028x; 1.0117x over previous
import jax
import jax.numpy as jnp
from jax import lax
from jax.experimental import pallas as pl
from jax.experimental.pallas import tpu as pltpu
from jax.experimental.pallas import tpu_sc as plsc

_N = 10000
_E = 320000
_D = 128
_NC = 2
_NS = 16
_NW = _NC * _NS
_C = 128
_CHUNKS = 80
_EPW = _C * _CHUNKS
_EPAD = _EPW * _NW
_PAD = _EPAD - _E
_NACC = 10240
_RPT = _NACC // _NS

_mesh = plsc.VectorSubcoreMesh(
    core_axis_name="c", subcore_axis_name="s", num_cores=_NC, num_subcores=_NS
)


def _fill2d(ref, rows, cols, val):
    def body(i, c):
        for k in range(cols // 16):
            ref[i, pl.ds(16 * k, 16)] = jnp.full((16,), val, jnp.float32)
        return c
    lax.fori_loop(0, rows, body, 0)


def _fill1d(ref, n, val):
    def body(i, c):
        ref[pl.ds(i * 16, 16)] = jnp.full((16,), val, jnp.float32)
        return c
    lax.fori_loop(0, n // 16, body, 0)


def _zero_acc(acc_sh, buf, sem, z0):
    for k in range(5):
        off = pl.multiple_of(z0 + k * 128, 8)
        pltpu.async_copy(buf, acc_sh.at[pl.ds(off, 128)], sem)
    for k in range(5):
        pltpu.make_async_copy(buf, acc_sh.at[pl.ds(z0, 128)], sem).wait()


def _copy_out(acc_sh, bufs, sems, out, cid, z0):
    offs = [pl.multiple_of(z0 + k * 128, 8) for k in range(5)]
    h1sem = [sems[0], sems[1]]
    h2sem = [sems[2], sems[3]]
    pltpu.async_copy(acc_sh.at[pl.ds(offs[0], 128)], bufs[0], h1sem[0])
    pltpu.async_copy(acc_sh.at[pl.ds(offs[1], 128)], bufs[1], h1sem[1])
    for k in range(5):
        p = k % 2
        pltpu.make_async_copy(acc_sh.at[pl.ds(offs[k], 128)], bufs[p], h1sem[p]).wait()
        pltpu.async_copy(bufs[p], out.at[cid, pl.ds(offs[k], 128)], h2sem[p])
        if k + 2 < 5:
            pltpu.make_async_copy(bufs[p], out.at[cid, pl.ds(offs[k], 128)], h2sem[p]).wait()
            pltpu.async_copy(acc_sh.at[pl.ds(offs[k + 2], 128)], bufs[p], h1sem[p])
    for k in (3, 4):
        p = k % 2
        pltpu.make_async_copy(bufs[p], out.at[cid, pl.ds(offs[k], 128)], h2sem[p]).wait()


def _gather_scatter_loop(table, src3, dst3, acc_sh, wid,
                         s0, s1, dst_a, rows_a, rows_b, gA, gB,
                         deg=None):
    pltpu.sync_copy(dst3.at[wid], dst_a)
    if deg is not None:
        deg_sh, ones_v = deg
    pltpu.sync_copy(src3.at[wid, 0], s0)
    pltpu.async_copy(table.at[s0], rows_a, gA)

    def pair(jj, c):
        j0 = 2 * jj
        pltpu.sync_copy(src3.at[wid, j0 + 1], s1)
        pltpu.async_copy(table.at[s1], rows_b, gB)
        pltpu.make_async_copy(table.at[pl.ds(0, _C)], rows_a, gA).wait()
        pltpu.sync_copy(rows_a, acc_sh.at[dst_a.at[j0]], add=True)
        if deg is not None:
            pltpu.sync_copy(ones_v, deg_sh.at[dst_a.at[j0]], add=True)

        @pl.when(jj + 1 < _CHUNKS // 2)
        def _():
            pltpu.sync_copy(src3.at[wid, j0 + 2], s0)
            pltpu.async_copy(table.at[s0], rows_a, gA)

        pltpu.make_async_copy(table.at[pl.ds(0, _C)], rows_b, gB).wait()
        pltpu.sync_copy(rows_b, acc_sh.at[dst_a.at[j0 + 1]], add=True)
        if deg is not None:
            pltpu.sync_copy(ones_v, deg_sh.at[dst_a.at[j0 + 1]], add=True)
        return c

    lax.fori_loop(0, _CHUNKS // 2, pair, 0)


def _agg_deg_body(table, src3, dst3, parts, degp,
                  s0, s1, dst_a, rows_a, rows_b, ones_v, vec_v,
                  acc_sh, deg_sh, gA, gB, sA, sB):
    cid = lax.axis_index("c")
    sid = lax.axis_index("s")
    wid = cid * _NS + sid
    z0 = pl.multiple_of(sid * _RPT, 8)
    _fill2d(rows_a, 128, _D, 0.0)
    _zero_acc(acc_sh, rows_a, gA, z0)
    _fill1d(vec_v, _RPT, 0.0)
    pltpu.sync_copy(vec_v, deg_sh.at[pl.ds(z0, _RPT)])
    _fill1d(ones_v, _C, 1.0)
    plsc.subcore_barrier()

    _gather_scatter_loop(table, src3, dst3, acc_sh, wid,
                         s0, s1, dst_a, rows_a, rows_b, gA, gB,
                         deg=(deg_sh, ones_v))
    plsc.subcore_barrier()

    _copy_out(acc_sh, (rows_a, rows_b), (gA, gB, sA, sB), parts, cid, z0)
    pltpu.sync_copy(deg_sh.at[pl.ds(z0, _RPT)], vec_v)
    pltpu.sync_copy(vec_v, degp.at[cid, pl.ds(z0, _RPT)])


def _agg_body(table, src3, dst3, parts,
              s0, s1, dst_a, rows_a, rows_b, acc_sh, gA, gB, sA, sB):
    cid = lax.axis_index("c")
    sid = lax.axis_index("s")
    wid = cid * _NS + sid
    z0 = pl.multiple_of(sid * _RPT, 8)
    _fill2d(rows_a, 128, _D, 0.0)
    _zero_acc(acc_sh, rows_a, gA, z0)
    plsc.subcore_barrier()

    _gather_scatter_loop(table, src3, dst3, acc_sh, wid,
                         s0, s1, dst_a, rows_a, rows_b, gA, gB)
    plsc.subcore_barrier()

    _copy_out(acc_sh, (rows_a, rows_b), (gA, gB, sA, sB), parts, cid, z0)


_agg_deg = pl.kernel(
    _agg_deg_body,
    out_type=(
        jax.ShapeDtypeStruct((_NC, _NACC, _D), jnp.float32),
        jax.ShapeDtypeStruct((_NC, _NACC), jnp.float32),
    ),
    mesh=_mesh,
    scratch_types=[
        pltpu.VMEM((_C,), jnp.int32),
        pltpu.VMEM((_C,), jnp.int32),
        pltpu.VMEM((_CHUNKS, _C), jnp.int32),
        pltpu.VMEM((_C, _D), jnp.float32),
        pltpu.VMEM((_C, _D), jnp.float32),
        pltpu.VMEM((_C,), jnp.float32),
        pltpu.VMEM((_RPT,), jnp.float32),
        pltpu.VMEM_SHARED((_NACC, _D), jnp.float32),
        pltpu.VMEM_SHARED((_NACC,), jnp.float32),
        pltpu.SemaphoreType.DMA,
        pltpu.SemaphoreType.DMA,
        pltpu.SemaphoreType.DMA,
        pltpu.SemaphoreType.DMA,
    ],
)

_agg = pl.kernel(
    _agg_body,
    out_type=jax.ShapeDtypeStruct((_NC, _NACC, _D), jnp.float32),
    mesh=_mesh,
    scratch_types=[
        pltpu.VMEM((_C,), jnp.int32),
        pltpu.VMEM((_C,), jnp.int32),
        pltpu.VMEM((_CHUNKS, _C), jnp.int32),
        pltpu.VMEM((_C, _D), jnp.float32),
        pltpu.VMEM((_C, _D), jnp.float32),
        pltpu.VMEM_SHARED((_NACC, _D), jnp.float32),
        pltpu.SemaphoreType.DMA,
        pltpu.SemaphoreType.DMA,
        pltpu.SemaphoreType.DMA,
        pltpu.SemaphoreType.DMA,
    ],
)


_R = 2000
_G = _N // _R

_DN = (((1,), (1,)), ((), ()))


def _dense_ln_body(parts, deg, h, Wl, bl, Wr, g, b, out):
    p = parts[0] + parts[1]
    agg = p / jnp.maximum(deg[...], 1.0)
    t = (lax.dot_general(agg, Wl[...], _DN, preferred_element_type=jnp.float32)
         + bl[...]
         + lax.dot_general(h[...], Wr[...], _DN, preferred_element_type=jnp.float32))
    mu = jnp.mean(t, axis=-1, keepdims=True)
    var = jnp.mean((t - mu) ** 2, axis=-1, keepdims=True)
    t = (t - mu) / jnp.sqrt(var + 1e-5) * g[...] + b[...]
    out[...] = jnp.maximum(t, 0.0) + h[...]


def _dense_fin_body(parts, deg, h, Wl, bl, Wr, out):
    p = parts[0] + parts[1]
    agg = p / jnp.maximum(deg[...], 1.0)
    out[...] = (lax.dot_general(agg, Wl[...], _DN, preferred_element_type=jnp.float32)
                + bl[...]
                + lax.dot_general(h[...], Wr[...], _DN, preferred_element_type=jnp.float32))


_spec_parts = pl.BlockSpec((_NC, _R, _D), lambda i: (0, i, 0))
_spec_deg = pl.BlockSpec((_R, 1), lambda i: (i, 0))
_spec_rows = pl.BlockSpec((_R, _D), lambda i: (i, 0))
_spec_w = pl.BlockSpec((_D, _D), lambda i: (0, 0))
_spec_v = pl.BlockSpec((1, _D), lambda i: (0, 0))

_dense_ln = pl.pallas_call(
    _dense_ln_body,
    grid=(_G,),
    in_specs=[_spec_parts, _spec_deg, _spec_rows, _spec_w, _spec_v,
              _spec_w, _spec_v, _spec_v],
    out_specs=_spec_rows,
    out_shape=jax.ShapeDtypeStruct((_N, _D), jnp.float32),
)

_dense_fin = pl.pallas_call(
    _dense_fin_body,
    grid=(_G,),
    in_specs=[_spec_parts, _spec_deg, _spec_rows, _spec_w, _spec_v,
              _spec_w],
    out_specs=_spec_rows,
    out_shape=jax.ShapeDtypeStruct((_N, _D), jnp.float32),
)


def kernel(x, edge_index, Wl0, bl0, Wr0, Wl1, bl1, Wr1, Wl2, bl2, Wr2,
           g0, b0, g1, b1):
    src = edge_index[0]
    dst = edge_index[1]
    ar = jnp.arange(_PAD, dtype=jnp.int32)
    src3 = jnp.concatenate([src, (ar * 37) % _N]).reshape(_NW, _CHUNKS, _C)
    dst3 = jnp.concatenate([dst, _N + (ar % (_NACC - _N))]).reshape(_NW, _CHUNKS, _C)

    bl0r, bl1r, bl2r = (v.reshape(1, _D) for v in (bl0, bl1, bl2))
    g0r, b0r, g1r, b1r = (v.reshape(1, _D) for v in (g0, b0, g1, b1))

    parts0, degp = _agg_deg(x, src3, dst3)
    deg = (degp[0, :_N] + degp[1, :_N]).reshape(_N, 1)
    h1 = _dense_ln(parts0, deg, x, Wl0, bl0r, Wr0, g0r, b0r)
    parts1 = _agg(h1, src3, dst3)
    h2 = _dense_ln(parts1, deg, h1, Wl1, bl1r, Wr1, g1r, b1r)
    parts2 = _agg(h2, src3, dst3)
    return _dense_fin(parts2, deg, h2, Wl2, bl2r, Wr2)
